# baseline (speedup 1.0000x reference)
import jax
import jax.numpy as jnp
from jax import lax
from jax.experimental import pallas as pl
from jax.experimental.pallas import tpu as pltpu
from jax.experimental.pallas import tpu_sc as plsc

NC = 2
NS = 16
NW = NC * NS
LANES = 16

N_IMG = 16
CR = 8
CHUNK = CR * 512
ROWS_PER_W = 256
N_CHUNKS = ROWS_PER_W // CR
NVEC = CHUNK // LANES
NB = 2
BINS = 32
TABLE = BINS * BINS * BINS


def _body(pred_h, targ_h, style_h, hist_h, out_h,
          hist_v, rings, aws_v, aw_v, sems):
    wid = lax.axis_index("s") * NC + lax.axis_index("c")
    n = wid // 2
    half = wid % 2
    base = half * ROWS_PER_W

    pltpu.sync_copy(hist_h, hist_v)

    srcs = (style_h, pred_h, targ_h)

    def issue(chunk, k):
        rr = base + chunk * CR
        for a, s in enumerate(srcs):
            pltpu.async_copy(s.at[pl.ds(3 * n, 3), pl.ds(rr, CR), :],
                             rings[k].at[pl.ds(3 * a, 3)], sems[k])

    def drain(k):
        pltpu.make_async_copy(
            pred_h.at[pl.ds(0, 9), pl.ds(0, CR), :], rings[k], sems[k]).wait()

    def compute(k, acc):
        buf = rings[k]

        def vec(v, u, a_ws, a_w):
            i = v >> 5
            sl = pl.ds(((v & 31) + u) * LANES, LANES)
            b0 = ((buf[0, i, sl] * 255.0) * 0.125).astype(jnp.int32)
            b1 = ((buf[1, i, sl] * 255.0) * 0.125).astype(jnp.int32)
            b2 = ((buf[2, i, sl] * 255.0) * 0.125).astype(jnp.int32)
            idx = b0 * (BINS * BINS) + b1 * BINS + b2
            idx = lax.bitcast_convert_type(
                jnp.minimum(lax.bitcast_convert_type(idx, jnp.uint32),
                            jnp.uint32(TABLE - 1)), jnp.int32)
            w = plsc.load_gather(hist_v, [idx])
            d = (jnp.abs(buf[3, i, sl] - buf[6, i, sl])
                 + jnp.abs(buf[4, i, sl] - buf[7, i, sl])
                 + jnp.abs(buf[5, i, sl] - buf[8, i, sl]))
            return a_ws + w * d, a_w + w

        def body(v, c):
            return tuple(vec(v, u, *c[u]) for u in range(4))

        return plsc.parallel_loop(0, NVEC, 4, carry=acc)(body)

    zeros = jnp.zeros((LANES,), jnp.float32)
    acc = ((zeros, zeros),) * 4

    for k in range(NB):
        issue(k, k)

    def outer(j, acc):
        for k in range(NB):
            drain(k)
            acc = compute(k, acc)

            @pl.when(NB * j + k + NB < N_CHUNKS)
            def _():
                issue(NB * j + k + NB, k)
        return acc

    acc = lax.fori_loop(0, N_CHUNKS // NB, outer, acc)

    aws_v[...] = acc[0][0] + acc[1][0] + acc[2][0] + acc[3][0]
    aw_v[...] = acc[0][1] + acc[1][1] + acc[2][1] + acc[3][1]
    pltpu.sync_copy(aws_v, out_h.at[0, wid])
    pltpu.sync_copy(aw_v, out_h.at[1, wid])


@jax.jit
def _sc_partials(pred, targ, style, hist):
    mesh = plsc.VectorSubcoreMesh(core_axis_name="c", subcore_axis_name="s")
    f = pl.kernel(
        _body,
        out_type=jax.ShapeDtypeStruct((2, NW, LANES), jnp.float32),
        mesh=mesh,
        compiler_params=pltpu.CompilerParams(needs_layout_passes=False),
        scratch_types=(
            [pltpu.VMEM((TABLE,), jnp.float32)]
            + [[pltpu.VMEM((9, CR, 512), jnp.float32) for _ in range(NB)]]
            + [pltpu.VMEM((LANES,), jnp.float32) for _ in range(2)]
            + [[pltpu.SemaphoreType.DMA for _ in range(NB)]]
        ),
    )
    return f(pred, targ, style, hist)


def kernel(pred, target, style_inp, hist):
    pred3 = pred.reshape(N_IMG * 3, 512, 512)
    targ3 = target.reshape(N_IMG * 3, 512, 512)
    style3 = style_inp.reshape(N_IMG * 3, 512, 512)
    hist_f = hist.reshape(TABLE)
    parts = _sc_partials(pred3, targ3, style3, hist_f)
    num = parts[0].sum()
    den = parts[1].sum() * 3.0
    return num / den


# speedup vs baseline: 1.1111x; 1.1111x over previous
import jax
import jax.numpy as jnp
from jax import lax
from jax.experimental import pallas as pl
from jax.experimental.pallas import tpu as pltpu
from jax.experimental.pallas import tpu_sc as plsc

NC = 2
NS = 16
NW = NC * NS
LANES = 16

N_IMG = 16
CR = 4
CHUNK = CR * 512
ROWS_PER_W = 256
N_CHUNKS = ROWS_PER_W // CR
NVEC = CHUNK // LANES
NB = 5
BINS = 32
TABLE = BINS * BINS * BINS


def _body(pred_h, targ_h, style_h, hist_h, out_h,
          hist_v, rings, aws_v, aw_v, sems):
    wid = lax.axis_index("s") * NC + lax.axis_index("c")
    n = wid // 2
    half = wid % 2
    base = half * ROWS_PER_W

    pltpu.sync_copy(hist_h, hist_v)

    srcs = (style_h, pred_h, targ_h)

    def issue(chunk, k):
        rr = base + chunk * CR
        for a, s in enumerate(srcs):
            pltpu.async_copy(s.at[pl.ds(3 * n, 3), pl.ds(rr, CR), :],
                             rings[k].at[pl.ds(3 * a, 3)], sems[k])

    def drain(k):
        pltpu.make_async_copy(
            pred_h.at[pl.ds(0, 9), pl.ds(0, CR), :], rings[k], sems[k]).wait()

    def compute(k, acc):
        buf = rings[k]

        def vec(v, u, a_ws, a_w):
            i = v >> 5
            sl = pl.ds(((v & 31) + u) * LANES, LANES)
            b0 = ((buf[0, i, sl] * 255.0) * 0.125).astype(jnp.int32)
            b1 = ((buf[1, i, sl] * 255.0) * 0.125).astype(jnp.int32)
            b2 = ((buf[2, i, sl] * 255.0) * 0.125).astype(jnp.int32)
            idx = b0 * (BINS * BINS) + b1 * BINS + b2
            idx = lax.bitcast_convert_type(
                jnp.minimum(lax.bitcast_convert_type(idx, jnp.uint32),
                            jnp.uint32(TABLE - 1)), jnp.int32)
            w = plsc.load_gather(hist_v, [idx])
            d = (jnp.abs(buf[3, i, sl] - buf[6, i, sl])
                 + jnp.abs(buf[4, i, sl] - buf[7, i, sl])
                 + jnp.abs(buf[5, i, sl] - buf[8, i, sl]))
            return a_ws + w * d, a_w + w

        def body(v, c):
            return tuple(vec(v, u, *c[u]) for u in range(4))

        return plsc.parallel_loop(0, NVEC, 4, carry=acc)(body)

    zeros = jnp.zeros((LANES,), jnp.float32)
    acc = ((zeros, zeros),) * 4

    for k in range(NB):
        issue(k, k)

    def outer(j, acc):
        for k in range(NB):
            drain(k)
            acc = compute(k, acc)

            @pl.when(NB * j + k + NB < N_CHUNKS)
            def _():
                issue(NB * j + k + NB, k)
        return acc

    acc = lax.fori_loop(0, N_CHUNKS // NB, outer, acc)

    aws_v[...] = acc[0][0] + acc[1][0] + acc[2][0] + acc[3][0]
    aw_v[...] = acc[0][1] + acc[1][1] + acc[2][1] + acc[3][1]
    pltpu.sync_copy(aws_v, out_h.at[0, wid])
    pltpu.sync_copy(aw_v, out_h.at[1, wid])


@jax.jit
def _sc_partials(pred, targ, style, hist):
    mesh = plsc.VectorSubcoreMesh(core_axis_name="c", subcore_axis_name="s")
    f = pl.kernel(
        _body,
        out_type=jax.ShapeDtypeStruct((2, NW, LANES), jnp.float32),
        mesh=mesh,
        compiler_params=pltpu.CompilerParams(needs_layout_passes=False),
        scratch_types=(
            [pltpu.VMEM((TABLE,), jnp.float32)]
            + [[pltpu.VMEM((9, CR, 512), jnp.float32) for _ in range(NB)]]
            + [pltpu.VMEM((LANES,), jnp.float32) for _ in range(2)]
            + [[pltpu.SemaphoreType.DMA for _ in range(NB)]]
        ),
    )
    return f(pred, targ, style, hist)


def kernel(pred, target, style_inp, hist):
    pred3 = pred.reshape(N_IMG * 3, 512, 512)
    targ3 = target.reshape(N_IMG * 3, 512, 512)
    style3 = style_inp.reshape(N_IMG * 3, 512, 512)
    hist_f = hist.reshape(TABLE)
    parts = _sc_partials(pred3, targ3, style3, hist_f)
    num = parts[0].sum()
    den = parts[1].sum() * 3.0
    return num / den
